# baseline (speedup 1.0000x reference)
import functools

import jax
import jax.numpy as jnp
from jax.experimental import pallas as pl
from jax.experimental.pallas import tpu as pltpu

_B = 16
_TB = 4096
_C = 128


def _mipl_body(cu_ref, xt_ref, w_ref,
               out_ref, m_ref, s_ref, n_ref, *, nblocks, tb, nseg, ps):
    step = pl.program_id(0)

    @pl.when(step == 0)
    def _init():
        m_ref[...] = jnp.full_like(m_ref, -1e30)
        s_ref[...] = jnp.zeros_like(s_ref)
        n_ref[...] = jnp.zeros_like(n_ref)

    xt = xt_ref[...]
    d = xt.shape[0]
    w2 = w_ref[...]
    zpad = jnp.zeros((d, _C - ps), jnp.float32)
    bu = jnp.concatenate([w2[:, :ps], zpad], axis=1)
    bz = jnp.concatenate([w2[:, ps:], zpad], axis=1)
    msq = jnp.mean(bz * bz, axis=0, keepdims=True)
    eta = bz * jnp.where(msq > 0, jax.lax.rsqrt(msq), 0.0)
    cdims = (((0,), (0,)), ((), ()))
    xw = jax.lax.dot_general(bu, xt, cdims,
                             preferred_element_type=jnp.float32)
    xv = jax.lax.dot_general(eta, xt, cdims,
                             preferred_element_type=jnp.float32)

    start = step * tb
    gidx = start + jax.lax.broadcasted_iota(jnp.int32, (nseg, tb), 1)
    lo_col = jnp.concatenate(
        [jnp.full((1, 1), cu_ref[b], jnp.int32) for b in range(nseg)], axis=0)
    hi_col = jnp.concatenate(
        [jnp.full((1, 1), cu_ref[b + 1], jnp.int32) for b in range(nseg)],
        axis=0)
    oh = ((gidx >= lo_col) & (gidx < hi_col)).astype(jnp.float32)

    bmax = jnp.max(xw, axis=1, keepdims=True)
    cols = []
    for b in range(nseg):
        hit = (cu_ref[b] < start + tb) & (cu_ref[b + 1] > start)
        cols.append(jnp.where(hit, bmax, jnp.full_like(bmax, -1e30)))
    m_blk = jnp.concatenate(cols, axis=1)
    m_old = m_ref[...]
    m_new = jnp.maximum(m_old, m_blk)
    scale = jnp.exp(m_old - m_new)
    m_tok = jnp.dot(m_new, oh, preferred_element_type=jnp.float32)
    e = jnp.exp(xw - m_tok)
    ep = jnp.concatenate([e, e * xv], axis=0)
    sn_add = jax.lax.dot_general(ep, oh, (((1,), (1,)), ((), ())),
                                 preferred_element_type=jnp.float32)
    m_ref[...] = m_new
    s_new = s_ref[...] * scale + sn_add[:_C, :]
    n_new = n_ref[...] * scale + sn_add[_C:, :]
    s_ref[...] = s_new
    n_ref[...] = n_new

    @pl.when(step == nblocks - 1)
    def _fin():
        s_t = jnp.swapaxes(s_new, 0, 1)
        n_t = jnp.swapaxes(n_new, 0, 1)
        z = jnp.where(s_t > 0, n_t / s_t, 0.0)
        bb = jnp.sqrt(msq)
        mean = jnp.mean(z, axis=0, keepdims=True)
        var = jnp.sum((z - mean) ** 2, axis=0, keepdims=True) / (nseg - 1)
        std = jnp.sqrt(var)
        std = jnp.where(jnp.isnan(std), 1.0, std)
        res = bb * (z - mean) / std
        out_ref[...] = res[:, :ps].reshape(out_ref.shape)


@functools.partial(jax.jit, static_argnums=(3, 4))
def _run(cu, x, w2, p, s):
    t, d = x.shape
    ps = p * s
    nblocks = t // _TB
    xt = jnp.transpose(x)
    body = functools.partial(_mipl_body, nblocks=nblocks, tb=_TB, nseg=_B,
                             ps=ps)
    return pl.pallas_call(
        body,
        grid=(nblocks,),
        in_specs=[
            pl.BlockSpec(memory_space=pltpu.SMEM),
            pl.BlockSpec((d, _TB), lambda i: (0, i)),
            pl.BlockSpec((d, 2 * ps), lambda i: (0, 0)),
        ],
        out_specs=pl.BlockSpec((_B, p, s), lambda i: (0, 0, 0)),
        out_shape=jax.ShapeDtypeStruct((_B, p, s), jnp.float32),
        scratch_shapes=[pltpu.VMEM((_C, _B), jnp.float32)] * 3,
        compiler_params=pltpu.CompilerParams(
            dimension_semantics=("arbitrary",)),
    )(cu, xt, w2)


def kernel(x, segment_ids, cu_seqlens, beta_u, beta_z):
    d, p, s = beta_u.shape
    w2 = jnp.concatenate([beta_u.reshape(d, p * s),
                          beta_z.reshape(d, p * s)], axis=1)
    return _run(cu_seqlens, x, w2, p, s)


# speedup vs baseline: 1.4984x; 1.4984x over previous
import functools

import jax
import jax.numpy as jnp
from jax.experimental import pallas as pl
from jax.experimental.pallas import tpu as pltpu

_B = 16
_TB = 8192


def _mipl_body(cu_ref, xt_ref, w_ref,
               out_ref, m_ref, s_ref, n_ref, *, nblocks, tb, nseg, ps):
    step = pl.program_id(0)

    @pl.when(step == 0)
    def _init():
        m_ref[...] = jnp.full_like(m_ref, -1e30)
        s_ref[...] = jnp.zeros_like(s_ref)
        n_ref[...] = jnp.zeros_like(n_ref)

    xt = xt_ref[...]
    w2 = w_ref[...]
    bu = w2[:ps, :]
    bz = w2[ps:, :]
    msq = jnp.mean(bz * bz, axis=1, keepdims=True)
    eta = bz * jax.lax.rsqrt(msq)
    cdims = (((1,), (0,)), ((), ()))
    xw = jax.lax.dot_general(bu, xt, cdims,
                             preferred_element_type=jnp.float32)
    xv = jax.lax.dot_general(eta, xt, cdims,
                             preferred_element_type=jnp.float32)

    start = step * tb
    gidx = start + jax.lax.broadcasted_iota(jnp.int32, (nseg, tb), 1)
    lo_col = jnp.concatenate(
        [jnp.full((1, 1), cu_ref[b], jnp.int32) for b in range(nseg)], axis=0)
    hi_col = jnp.concatenate(
        [jnp.full((1, 1), cu_ref[b + 1], jnp.int32) for b in range(nseg)],
        axis=0)
    oh = ((gidx >= lo_col) & (gidx < hi_col)).astype(jnp.float32)

    bmax = jnp.max(xw, axis=1, keepdims=True)
    cols = []
    for b in range(nseg):
        hit = (cu_ref[b] < start + tb) & (cu_ref[b + 1] > start)
        cols.append(jnp.where(hit, bmax, jnp.full_like(bmax, -1e30)))
    m_blk = jnp.concatenate(cols, axis=1)
    m_old = m_ref[...]
    m_new = jnp.maximum(m_old, m_blk)
    scale = jnp.exp(m_old - m_new)
    m_tok = jnp.dot(m_new, oh, preferred_element_type=jnp.float32)
    e = jnp.exp(xw - m_tok)
    ep = jnp.concatenate([e, e * xv], axis=0)
    sn_add = jax.lax.dot_general(ep, oh, (((1,), (1,)), ((), ())),
                                 preferred_element_type=jnp.float32)
    m_ref[...] = m_new
    s_new = s_ref[...] * scale + sn_add[:ps, :]
    n_new = n_ref[...] * scale + sn_add[ps:, :]
    s_ref[...] = s_new
    n_ref[...] = n_new

    @pl.when(step == nblocks - 1)
    def _fin():
        s_t = jnp.swapaxes(s_new, 0, 1)
        n_t = jnp.swapaxes(n_new, 0, 1)
        z = jnp.where(s_t > 0, n_t / s_t, 0.0)
        bb = jnp.sqrt(jnp.swapaxes(msq, 0, 1))
        mean = jnp.mean(z, axis=0, keepdims=True)
        var = jnp.sum((z - mean) ** 2, axis=0, keepdims=True) / (nseg - 1)
        std = jnp.sqrt(var)
        std = jnp.where(jnp.isnan(std), 1.0, std)
        res = bb * (z - mean) / std
        out_ref[...] = res.reshape(out_ref.shape)


@functools.partial(jax.jit, static_argnums=(3, 4))
def _run(cu, x, w2, p, s):
    t, d = x.shape
    ps = p * s
    nblocks = t // _TB
    xt = jnp.transpose(x)
    body = functools.partial(_mipl_body, nblocks=nblocks, tb=_TB, nseg=_B,
                             ps=ps)
    return pl.pallas_call(
        body,
        grid=(nblocks,),
        in_specs=[
            pl.BlockSpec(memory_space=pltpu.SMEM),
            pl.BlockSpec((d, _TB), lambda i: (0, i)),
            pl.BlockSpec((2 * ps, d), lambda i: (0, 0)),
        ],
        out_specs=pl.BlockSpec((_B, p, s), lambda i: (0, 0, 0)),
        out_shape=jax.ShapeDtypeStruct((_B, p, s), jnp.float32),
        scratch_shapes=[pltpu.VMEM((ps, _B), jnp.float32)] * 3,
        compiler_params=pltpu.CompilerParams(
            dimension_semantics=("arbitrary",)),
    )(cu, xt, w2)


def kernel(x, segment_ids, cu_seqlens, beta_u, beta_z):
    d, p, s = beta_u.shape
    ps = p * s
    w2 = jnp.concatenate([beta_u.reshape(d, ps).T,
                          beta_z.reshape(d, ps).T], axis=0)
    return _run(cu_seqlens, x, w2, p, s)


# speedup vs baseline: 1.5291x; 1.0205x over previous
import functools

import jax
import jax.numpy as jnp
from jax.experimental import pallas as pl
from jax.experimental.pallas import tpu as pltpu

_B = 16
_TB = 8192


def _mipl_body(cu_ref, xt_ref, w_ref,
               out_ref, m_ref, s_ref, n_ref, *, nblocks, tb, nseg, ps):
    step = pl.program_id(0)

    @pl.when(step == 0)
    def _init():
        m_ref[...] = jnp.full_like(m_ref, -1e30)
        s_ref[...] = jnp.zeros_like(s_ref)
        n_ref[...] = jnp.zeros_like(n_ref)

    xt = xt_ref[...]
    w2 = w_ref[...]
    bu = w2[:ps, :]
    bz = w2[ps:, :]
    msq = jnp.mean(bz * bz, axis=1, keepdims=True)
    eta = bz * jax.lax.rsqrt(msq)
    cdims = (((1,), (0,)), ((), ()))
    xw = jax.lax.dot_general(bu, xt, cdims,
                             preferred_element_type=jnp.float32)
    xv = jax.lax.dot_general(eta, xt, cdims,
                             preferred_element_type=jnp.float32)

    start = step * tb
    gidx = start + jax.lax.broadcasted_iota(jnp.int32, (nseg, tb), 1)
    lo_col = jnp.concatenate(
        [jnp.full((1, 1), cu_ref[b], jnp.int32) for b in range(nseg)], axis=0)
    hi_col = jnp.concatenate(
        [jnp.full((1, 1), cu_ref[b + 1], jnp.int32) for b in range(nseg)],
        axis=0)
    oh = ((gidx >= lo_col) & (gidx < hi_col)).astype(jnp.float32)

    bmax = jnp.max(xw, axis=1, keepdims=True)
    cols = []
    for b in range(nseg):
        hit = (cu_ref[b] < start + tb) & (cu_ref[b + 1] > start)
        cols.append(jnp.where(hit, bmax, jnp.full_like(bmax, -1e30)))
    m_blk = jnp.concatenate(cols, axis=1)
    m_old = m_ref[...]
    m_new = jnp.maximum(m_old, m_blk)
    scale = jnp.exp(m_old - m_new)
    m_tok = jnp.dot(m_new, oh, preferred_element_type=jnp.float32)
    e = jnp.exp(xw - m_tok)
    rdims = (((1,), (1,)), ((), ()))
    s_add = jax.lax.dot_general(e, oh, rdims,
                                preferred_element_type=jnp.float32)
    n_add = jax.lax.dot_general(e * xv, oh, rdims,
                                preferred_element_type=jnp.float32)
    m_ref[...] = m_new
    s_new = s_ref[...] * scale + s_add
    n_new = n_ref[...] * scale + n_add
    s_ref[...] = s_new
    n_ref[...] = n_new

    @pl.when(step == nblocks - 1)
    def _fin():
        s_t = jnp.swapaxes(s_new, 0, 1)
        n_t = jnp.swapaxes(n_new, 0, 1)
        z = jnp.where(s_t > 0, n_t / s_t, 0.0)
        bb = jnp.sqrt(jnp.swapaxes(msq, 0, 1))
        mean = jnp.mean(z, axis=0, keepdims=True)
        var = jnp.sum((z - mean) ** 2, axis=0, keepdims=True) / (nseg - 1)
        std = jnp.sqrt(var)
        std = jnp.where(jnp.isnan(std), 1.0, std)
        res = bb * (z - mean) / std
        out_ref[...] = res.reshape(out_ref.shape)


@functools.partial(jax.jit, static_argnums=(3, 4))
def _run(cu, x, w2, p, s):
    t, d = x.shape
    ps = p * s
    nblocks = t // _TB
    xt = jnp.transpose(x)
    body = functools.partial(_mipl_body, nblocks=nblocks, tb=_TB, nseg=_B,
                             ps=ps)
    return pl.pallas_call(
        body,
        grid=(nblocks,),
        in_specs=[
            pl.BlockSpec(memory_space=pltpu.SMEM),
            pl.BlockSpec((d, _TB), lambda i: (0, i)),
            pl.BlockSpec((2 * ps, d), lambda i: (0, 0)),
        ],
        out_specs=pl.BlockSpec((_B, p, s), lambda i: (0, 0, 0)),
        out_shape=jax.ShapeDtypeStruct((_B, p, s), jnp.float32),
        scratch_shapes=[pltpu.VMEM((ps, _B), jnp.float32)] * 3,
        compiler_params=pltpu.CompilerParams(
            dimension_semantics=("arbitrary",)),
    )(cu, xt, w2)


def kernel(x, segment_ids, cu_seqlens, beta_u, beta_z):
    d, p, s = beta_u.shape
    ps = p * s
    w2 = jnp.concatenate([beta_u.reshape(d, ps).T,
                          beta_z.reshape(d, ps).T], axis=0)
    return _run(cu_seqlens, x, w2, p, s)
